# baseline (speedup 1.0000x reference)
import functools

import jax
import jax.numpy as jnp
from jax import lax
from jax.experimental import pallas as pl
from jax.experimental.pallas import tpu as pltpu
from jax.experimental.pallas import tpu_sc as plsc

N = 50000
E = 800000
D = 64
R = 10
HALF = 32

NS = 16
SUB = 80
NSUB = 5
C = SUB * NSUB
EPT = E // NS
NCH = EPT // C
RPT = N // NS
ZR = 25


def _spmm_body(x2, cin, rin, vin, cout, rout, vout, nin2, nout2,
               acc, cv, gv, vv, rv, G, zbuf, sem_in, sem_g, sem_s):
    core = lax.axis_index("c")
    sid = lax.axis_index("s")

    def _zb(r, _):
        z16 = jnp.zeros((16,), jnp.float32)
        zbuf[r, pl.ds(0, 16)] = z16
        zbuf[r, pl.ds(16, 16)] = z16
        return 0
    lax.fori_loop(0, ZR, _zb, 0)

    def run_pass(cols_hbm, rows_hbm, vals_hbm, out_hbm):
        def _zz(zg, _):
            cps = [pltpu.async_copy(
                zbuf, acc.at[pl.ds(sid * RPT + (zg * 5 + u) * ZR, ZR)], sem_s)
                for u in range(5)]
            for cp in cps:
                cp.wait()
            return 0
        lax.fori_loop(0, RPT // ZR // 5, _zz, 0)
        plsc.subcore_barrier()

        def issue_in(k, p, r):
            ebase = sid * EPT + k * C
            pltpu.async_copy(cols_hbm.at[pl.ds(ebase, C)], cv.at[p], sem_in)
            pltpu.async_copy(vals_hbm.at[pl.ds(ebase, C)], vv.at[p], sem_in)
            for j in range(NSUB):
                pltpu.async_copy(rows_hbm.at[pl.ds(ebase + j * SUB, SUB)],
                                 rv.at[r, j], sem_in)

        def wait_in(p, r):
            pltpu.make_async_copy(cols_hbm.at[pl.ds(0, C)],
                                  cv.at[p], sem_in).wait()
            pltpu.make_async_copy(vals_hbm.at[pl.ds(0, C)],
                                  vv.at[p], sem_in).wait()
            for j in range(NSUB):
                pltpu.make_async_copy(rows_hbm.at[pl.ds(0, SUB)],
                                      rv.at[r, j], sem_in).wait()

        def compute_gidx(p):
            def _gi(fg, _):
                o = fg * 16
                c16 = cv[p, pl.ds(o, 16)]
                gv[p, pl.ds(o, 16)] = c16 + c16 + core
                return 0
            lax.fori_loop(0, C // 16, _gi, 0)

        def issue_gather(p):
            for j in range(NSUB):
                pltpu.async_copy(x2.at[gv.at[p, pl.ds(j * SUB, SUB)]],
                                 G.at[p, pl.ds(j * SUB, SUB)], sem_g)

        def wait_gather(p):
            for j in range(NSUB):
                pltpu.make_async_copy(
                    x2.at[gv.at[p, pl.ds(j * SUB, SUB)]],
                    G.at[p, pl.ds(j * SUB, SUB)], sem_g).wait()

        def scale(p):
            def _sc(fg, _):
                o = fg * 16
                v16 = vv[p, pl.ds(o, 16)]
                for e in range(16):
                    row = o + e
                    s = v16[e]
                    G[p, row, pl.ds(0, 16)] = G[p, row, pl.ds(0, 16)] * s
                    G[p, row, pl.ds(16, 16)] = G[p, row, pl.ds(16, 16)] * s
                return 0
            lax.fori_loop(0, C // 16, _sc, 0)

        def issue_scatter(p, r):
            for j in range(NSUB):
                pltpu.async_copy(G.at[p, pl.ds(j * SUB, SUB)],
                                 acc.at[rv.at[r, j]], sem_s, add=True)

        def drain_scatter(p, r):
            for j in range(NSUB):
                pltpu.make_async_copy(G.at[p, pl.ds(j * SUB, SUB)],
                                      acc.at[rv.at[r, j]], sem_s).wait()

        issue_in(0, 0, 0)
        wait_in(0, 0)
        compute_gidx(0)
        issue_gather(0)
        issue_in(1, 1, 1)

        def quad(i4, _):
            for b4 in range(4):
                k = i4 * 4 + b4
                p = b4 % 2
                pn = 1 - p
                rn = (b4 + 1) % 4
                rp = (b4 - 1) % 4
                r2 = (b4 + 2) % 4
                wait_gather(p)

                @pl.when(k < NCH - 1)
                def _():
                    wait_in(pn, rn)
                    compute_gidx(pn)

                @pl.when(k > 0)
                def _():
                    drain_scatter(pn, rp)

                @pl.when(k < NCH - 1)
                def _():
                    issue_gather(pn)

                scale(p)
                issue_scatter(p, b4)

                @pl.when(k < NCH - 2)
                def _():
                    issue_in(k + 2, p, r2)
            return 0

        lax.fori_loop(0, NCH // 4, quad, 0)
        wait_gather(0)
        drain_scatter(1, 3)
        scale(0)
        issue_scatter(0, 0)
        drain_scatter(0, 0)

        plsc.subcore_barrier()
        pltpu.sync_copy(acc.at[pl.ds(sid * RPT, RPT)],
                        out_hbm.at[core, pl.ds(sid * RPT, RPT)])
        plsc.subcore_barrier()

    run_pass(cin, rin, vin, nin2)
    run_pass(cout, rout, vout, nout2)


@jax.jit
def _spmm2(x2, cin, rin, vin, cout, rout, vout):
    mesh = plsc.VectorSubcoreMesh(core_axis_name="c", subcore_axis_name="s")
    f = pl.kernel(
        _spmm_body,
        out_type=(jax.ShapeDtypeStruct((2, N, HALF), jnp.float32),
                  jax.ShapeDtypeStruct((2, N, HALF), jnp.float32)),
        mesh=mesh,
        scratch_types=(
            pltpu.VMEM_SHARED((N, HALF), jnp.float32),
            pltpu.VMEM((2, C), jnp.int32),
            pltpu.VMEM((2, C), jnp.int32),
            pltpu.VMEM((2, C), jnp.float32),
            pltpu.VMEM((4, NSUB, SUB), jnp.int32),
            pltpu.VMEM((2, C, HALF), jnp.float32),
            pltpu.VMEM((ZR, HALF), jnp.float32),
            pltpu.SemaphoreType.DMA,
            pltpu.SemaphoreType.DMA,
            pltpu.SemaphoreType.DMA,
        ),
        compiler_params=pltpu.CompilerParams(use_tc_tiling_on_sc=False),
    )
    return f(x2, cin, rin, vin, cout, rout, vout)


BN = 2000


BX = 80000


def _split_body(idx_ref, r_ref, c_ref):
    r_ref[...] = idx_ref[0]
    c_ref[...] = idx_ref[1]


@jax.jit
def _split(idx):
    return pl.pallas_call(
        _split_body,
        grid=(1,),
        in_specs=[pl.BlockSpec((2, E), lambda i: (0, 0))],
        out_specs=[
            pl.BlockSpec((E,), lambda i: (0,)),
            pl.BlockSpec((E,), lambda i: (0,)),
        ],
        out_shape=[
            jax.ShapeDtypeStruct((E,), jnp.int32),
            jax.ShapeDtypeStruct((E,), jnp.int32),
        ],
    )(idx)


def _pack_body(x_ref, o_ref):
    o_ref[:, 0:D] = x_ref[pl.ds(0, BN // 2, 2), :]
    o_ref[:, D:2 * D] = x_ref[pl.ds(1, BN // 2, 2), :]


@jax.jit
def _pack(x):
    return pl.pallas_call(
        _pack_body,
        grid=(N // BN,),
        in_specs=[pl.BlockSpec((BN, D), lambda i: (i, 0))],
        out_specs=pl.BlockSpec((BN // 2, 2 * D), lambda i: (i, 0)),
        out_shape=jax.ShapeDtypeStruct((N // 2, 2 * D), jnp.float32),
    )(x)


def _layer_body(ego_ref, nin_ref, nout_ref, w_ref, ego_o_ref, emb_o_ref,
                scr_ref, nb_ref, sem):
    i = pl.program_id(0)
    cps = []
    for h in range(2):
        cps.append(pltpu.make_async_copy(
            nin_ref.at[h, pl.ds(i * BN, BN)], nb_ref.at[h], sem))
        cps.append(pltpu.make_async_copy(
            nout_ref.at[h, pl.ds(i * BN, BN)], nb_ref.at[2 + h], sem))
    for cp in cps:
        cp.start()
    scr_ref[pl.ds(0, BN // 2, 2), :] = ego_ref[:, 0:D]
    scr_ref[pl.ds(1, BN // 2, 2), :] = ego_ref[:, D:2 * D]
    ego = scr_ref[...]
    w = w_ref[...]
    acc = jnp.dot(ego, w[0:64], preferred_element_type=jnp.float32)
    for cp in cps:
        cp.wait()
    acc = acc + jnp.dot(nb_ref[0], w[64:96], preferred_element_type=jnp.float32)
    acc = acc + jnp.dot(nb_ref[1], w[96:128], preferred_element_type=jnp.float32)
    acc = acc + jnp.dot(nb_ref[2], w[128:160], preferred_element_type=jnp.float32)
    acc = acc + jnp.dot(nb_ref[3], w[160:192], preferred_element_type=jnp.float32)
    ego_n = jnp.where(acc >= 0, acc, 0.01 * acc)
    scr_ref[...] = ego_n
    ego_o_ref[:, 0:D] = scr_ref[pl.ds(0, BN // 2, 2), :]
    ego_o_ref[:, D:2 * D] = scr_ref[pl.ds(1, BN // 2, 2), :]
    nrm = jnp.sqrt(jnp.sum(ego_n * ego_n, axis=1, keepdims=True))
    emb_o_ref[...] = ego_n / jnp.maximum(nrm, 1e-12)


@jax.jit
def _layer(ego_p, nin2, nout2, w):
    return pl.pallas_call(
        _layer_body,
        grid=(N // BN,),
        in_specs=[
            pl.BlockSpec((BN // 2, 2 * D), lambda i: (i, 0)),
            pl.BlockSpec(memory_space=pltpu.MemorySpace.HBM),
            pl.BlockSpec(memory_space=pltpu.MemorySpace.HBM),
            pl.BlockSpec((3 * D, D), lambda i: (0, 0)),
        ],
        out_specs=[
            pl.BlockSpec((BN // 2, 2 * D), lambda i: (i, 0)),
            pl.BlockSpec((BN, D), lambda i: (i, 0)),
        ],
        out_shape=[
            jax.ShapeDtypeStruct((N // 2, 2 * D), jnp.float32),
            jax.ShapeDtypeStruct((N, D), jnp.float32),
        ],
        scratch_shapes=[pltpu.VMEM((BN, D), jnp.float32),
                        pltpu.VMEM((4, BN, HALF), jnp.float32),
                        pltpu.SemaphoreType.DMA],
    )(ego_p, nin2, nout2, w)


def _kg_body(attr_ref, e0_ref, e1_ref, e2_ref, out_ref):
    i = pl.program_id(0)
    a = attr_ref[...].reshape(BN, 1)
    lbl = lax.broadcasted_iota(jnp.int32, (BN, R), 1).astype(jnp.float32)
    oh = (a == lbl).astype(jnp.float32)
    dn = (((0,), (0,)), ((), ()))
    r0 = lax.dot_general(oh, e0_ref[...], dn, preferred_element_type=jnp.float32)
    r1 = lax.dot_general(oh, e1_ref[...], dn, preferred_element_type=jnp.float32)
    r2 = lax.dot_general(oh, e2_ref[...], dn, preferred_element_type=jnp.float32)
    contrib = jnp.concatenate([r0, r1, r2], axis=1)

    @pl.when(i == 0)
    def _():
        out_ref[...] = jnp.zeros_like(out_ref)

    out_ref[...] += contrib


@jax.jit
def _kg(attr_f, e0, e1, e2):
    return pl.pallas_call(
        _kg_body,
        grid=(N // BN,),
        in_specs=[
            pl.BlockSpec((1, 1, BN), lambda i: (i, 0, 0)),
            pl.BlockSpec((BN, D), lambda i: (i, 0)),
            pl.BlockSpec((BN, D), lambda i: (i, 0)),
            pl.BlockSpec((BN, D), lambda i: (i, 0)),
        ],
        out_specs=pl.BlockSpec((R, 3 * D), lambda i: (0, 0)),
        out_shape=jax.ShapeDtypeStruct((R, 3 * D), jnp.float32),
    )(attr_f, e0, e1, e2)


def kernel(edge_emb, W1, W2, Hin_idx, Hin_val, Hout_idx, Hout_val, edge_attr):
    rin, cin = _split(Hin_idx)
    rout, cout = _split(Hout_idx)

    ego0p = _pack(edge_emb)
    nin2, nout2 = _spmm2(ego0p.reshape(2 * N, HALF), cin, rin, Hin_val,
                         cout, rout, Hout_val)
    ego1p, emb1 = _layer(ego0p, nin2, nout2, W1)
    nin2b, nout2b = _spmm2(ego1p.reshape(2 * N, HALF), cin, rin, Hin_val,
                           cout, rout, Hout_val)
    _, emb2 = _layer(ego1p, nin2b, nout2b, W2)

    attr_f = edge_attr.astype(jnp.float32).reshape(N // BN, 1, BN)
    return _kg(attr_f, edge_emb, emb1, emb2)


# speedup vs baseline: 1.1350x; 1.1350x over previous
import functools

import jax
import jax.numpy as jnp
from jax import lax
from jax.experimental import pallas as pl
from jax.experimental.pallas import tpu as pltpu
from jax.experimental.pallas import tpu_sc as plsc

N = 50000
E = 800000
D = 64
R = 10
HALF = 32

NS = 16
SUB = 80
NSUB = 5
C = SUB * NSUB
EPT = E // NS
NCH = EPT // C
RPT = N // NS
ZR = 25


def _spmm_body(x2, cin, rin, vin, cout, rout, vout, nin2, nout2,
               acc, cv, gv, vv, rv, G, zbuf, sem_in, sem_g, sem_s):
    core = lax.axis_index("c")
    sid = lax.axis_index("s")

    def _zb(r, _):
        z16 = jnp.zeros((16,), jnp.float32)
        zbuf[r, pl.ds(0, 16)] = z16
        zbuf[r, pl.ds(16, 16)] = z16
        return 0
    lax.fori_loop(0, ZR, _zb, 0)

    def run_pass(cols_hbm, rows_hbm, vals_hbm, out_hbm):
        def _zz(zg, _):
            cps = [pltpu.async_copy(
                zbuf, acc.at[pl.ds(sid * RPT + (zg * 5 + u) * ZR, ZR)], sem_s)
                for u in range(5)]
            for cp in cps:
                cp.wait()
            return 0
        lax.fori_loop(0, RPT // ZR // 5, _zz, 0)
        plsc.subcore_barrier()

        def issue_in(k, p, r):
            ebase = sid * EPT + k * C
            pltpu.async_copy(cols_hbm.at[pl.ds(ebase, C)], cv.at[p], sem_in)
            pltpu.async_copy(vals_hbm.at[pl.ds(ebase, C)], vv.at[p], sem_in)
            for j in range(NSUB):
                pltpu.async_copy(rows_hbm.at[pl.ds(ebase + j * SUB, SUB)],
                                 rv.at[r, j], sem_in)

        def wait_in(p, r):
            pltpu.make_async_copy(cols_hbm.at[pl.ds(0, C)],
                                  cv.at[p], sem_in).wait()
            pltpu.make_async_copy(vals_hbm.at[pl.ds(0, C)],
                                  vv.at[p], sem_in).wait()
            for j in range(NSUB):
                pltpu.make_async_copy(rows_hbm.at[pl.ds(0, SUB)],
                                      rv.at[r, j], sem_in).wait()

        def compute_gidx(p):
            def _gi(f5, _):
                for g5 in range(5):
                    o = f5 * 80 + g5 * 16
                    c16 = cv[p, pl.ds(o, 16)]
                    gv[p, pl.ds(o, 16)] = c16 + c16 + core
                return 0
            lax.fori_loop(0, C // 80, _gi, 0)

        def issue_gather(p):
            for j in range(NSUB):
                pltpu.async_copy(x2.at[gv.at[p, pl.ds(j * SUB, SUB)]],
                                 G.at[p, pl.ds(j * SUB, SUB)], sem_g)

        def wait_gather(p):
            for j in range(NSUB):
                pltpu.make_async_copy(
                    x2.at[gv.at[p, pl.ds(j * SUB, SUB)]],
                    G.at[p, pl.ds(j * SUB, SUB)], sem_g).wait()

        def scale(p):
            def _sc(f5, _):
                for g5 in range(5):
                    o = f5 * 80 + g5 * 16
                    v16 = vv[p, pl.ds(o, 16)]
                    for e in range(16):
                        row = o + e
                        s = v16[e]
                        G[p, row, pl.ds(0, 16)] = G[p, row, pl.ds(0, 16)] * s
                        G[p, row, pl.ds(16, 16)] = G[p, row, pl.ds(16, 16)] * s
                return 0
            lax.fori_loop(0, C // 80, _sc, 0)

        def issue_scatter(p, r):
            for j in range(NSUB):
                pltpu.async_copy(G.at[p, pl.ds(j * SUB, SUB)],
                                 acc.at[rv.at[r, j]], sem_s, add=True)

        def drain_scatter(p, r):
            for j in range(NSUB):
                pltpu.make_async_copy(G.at[p, pl.ds(j * SUB, SUB)],
                                      acc.at[rv.at[r, j]], sem_s).wait()

        issue_in(0, 0, 0)
        wait_in(0, 0)
        compute_gidx(0)
        issue_gather(0)
        issue_in(1, 1, 1)

        def quad(i4, _):
            for b4 in range(4):
                k = i4 * 4 + b4
                p = b4 % 2
                pn = 1 - p
                rn = (b4 + 1) % 4
                rp = (b4 - 1) % 4
                r2 = (b4 + 2) % 4
                wait_gather(p)

                @pl.when(k < NCH - 1)
                def _():
                    wait_in(pn, rn)
                    compute_gidx(pn)

                @pl.when(k > 0)
                def _():
                    drain_scatter(pn, rp)

                @pl.when(k < NCH - 1)
                def _():
                    issue_gather(pn)

                scale(p)
                issue_scatter(p, b4)

                @pl.when(k < NCH - 2)
                def _():
                    issue_in(k + 2, p, r2)
            return 0

        lax.fori_loop(0, NCH // 4, quad, 0)
        wait_gather(0)
        drain_scatter(1, 3)
        scale(0)
        issue_scatter(0, 0)
        drain_scatter(0, 0)

        plsc.subcore_barrier()
        pltpu.sync_copy(acc.at[pl.ds(sid * RPT, RPT)],
                        out_hbm.at[core, pl.ds(sid * RPT, RPT)])
        plsc.subcore_barrier()

    run_pass(cin, rin, vin, nin2)
    run_pass(cout, rout, vout, nout2)


@jax.jit
def _spmm2(x2, cin, rin, vin, cout, rout, vout):
    mesh = plsc.VectorSubcoreMesh(core_axis_name="c", subcore_axis_name="s")
    f = pl.kernel(
        _spmm_body,
        out_type=(jax.ShapeDtypeStruct((2, N, HALF), jnp.float32),
                  jax.ShapeDtypeStruct((2, N, HALF), jnp.float32)),
        mesh=mesh,
        scratch_types=(
            pltpu.VMEM_SHARED((N, HALF), jnp.float32),
            pltpu.VMEM((2, C), jnp.int32),
            pltpu.VMEM((2, C), jnp.int32),
            pltpu.VMEM((2, C), jnp.float32),
            pltpu.VMEM((4, NSUB, SUB), jnp.int32),
            pltpu.VMEM((2, C, HALF), jnp.float32),
            pltpu.VMEM((ZR, HALF), jnp.float32),
            pltpu.SemaphoreType.DMA,
            pltpu.SemaphoreType.DMA,
            pltpu.SemaphoreType.DMA,
        ),
        compiler_params=pltpu.CompilerParams(use_tc_tiling_on_sc=False),
    )
    return f(x2, cin, rin, vin, cout, rout, vout)


BN = 2000


BX = 80000


def _split_body(idx_ref, r_ref, c_ref):
    r_ref[...] = idx_ref[0]
    c_ref[...] = idx_ref[1]


@jax.jit
def _split(idx):
    return pl.pallas_call(
        _split_body,
        grid=(1,),
        in_specs=[pl.BlockSpec((2, E), lambda i: (0, 0))],
        out_specs=[
            pl.BlockSpec((E,), lambda i: (0,)),
            pl.BlockSpec((E,), lambda i: (0,)),
        ],
        out_shape=[
            jax.ShapeDtypeStruct((E,), jnp.int32),
            jax.ShapeDtypeStruct((E,), jnp.int32),
        ],
    )(idx)


def _pack_body(x_ref, o_ref):
    o_ref[:, 0:D] = x_ref[pl.ds(0, BN // 2, 2), :]
    o_ref[:, D:2 * D] = x_ref[pl.ds(1, BN // 2, 2), :]


@jax.jit
def _pack(x):
    return pl.pallas_call(
        _pack_body,
        grid=(N // BN,),
        in_specs=[pl.BlockSpec((BN, D), lambda i: (i, 0))],
        out_specs=pl.BlockSpec((BN // 2, 2 * D), lambda i: (i, 0)),
        out_shape=jax.ShapeDtypeStruct((N // 2, 2 * D), jnp.float32),
    )(x)


def _layer_body(ego_ref, nin_ref, nout_ref, w_ref, ego_o_ref, emb_o_ref,
                scr_ref):
    scr_ref[pl.ds(0, BN // 2, 2), :] = ego_ref[:, 0:D]
    scr_ref[pl.ds(1, BN // 2, 2), :] = ego_ref[:, D:2 * D]
    ego = scr_ref[...]
    w = w_ref[...]
    acc = jnp.dot(ego, w[0:64], preferred_element_type=jnp.float32)
    acc = acc + jnp.dot(nin_ref[0], w[64:96], preferred_element_type=jnp.float32)
    acc = acc + jnp.dot(nin_ref[1], w[96:128], preferred_element_type=jnp.float32)
    acc = acc + jnp.dot(nout_ref[0], w[128:160], preferred_element_type=jnp.float32)
    acc = acc + jnp.dot(nout_ref[1], w[160:192], preferred_element_type=jnp.float32)
    ego_n = jnp.where(acc >= 0, acc, 0.01 * acc)
    scr_ref[...] = ego_n
    ego_o_ref[:, 0:D] = scr_ref[pl.ds(0, BN // 2, 2), :]
    ego_o_ref[:, D:2 * D] = scr_ref[pl.ds(1, BN // 2, 2), :]
    nrm = jnp.sqrt(jnp.sum(ego_n * ego_n, axis=1, keepdims=True))
    emb_o_ref[...] = ego_n / jnp.maximum(nrm, 1e-12)


@jax.jit
def _layer(ego_p, nin2, nout2, w):
    return pl.pallas_call(
        _layer_body,
        grid=(N // BN,),
        in_specs=[
            pl.BlockSpec((BN // 2, 2 * D), lambda i: (i, 0)),
            pl.BlockSpec((2, BN, HALF), lambda i: (0, i, 0)),
            pl.BlockSpec((2, BN, HALF), lambda i: (0, i, 0)),
            pl.BlockSpec((3 * D, D), lambda i: (0, 0)),
        ],
        out_specs=[
            pl.BlockSpec((BN // 2, 2 * D), lambda i: (i, 0)),
            pl.BlockSpec((BN, D), lambda i: (i, 0)),
        ],
        out_shape=[
            jax.ShapeDtypeStruct((N // 2, 2 * D), jnp.float32),
            jax.ShapeDtypeStruct((N, D), jnp.float32),
        ],
        scratch_shapes=[pltpu.VMEM((BN, D), jnp.float32)],
    )(ego_p, nin2, nout2, w)


def _kg_body(attr_ref, e0_ref, e1_ref, e2_ref, out_ref):
    i = pl.program_id(0)
    a = attr_ref[...].reshape(BN, 1)
    lbl = lax.broadcasted_iota(jnp.int32, (BN, R), 1).astype(jnp.float32)
    oh = (a == lbl).astype(jnp.float32)
    dn = (((0,), (0,)), ((), ()))
    r0 = lax.dot_general(oh, e0_ref[...], dn, preferred_element_type=jnp.float32)
    r1 = lax.dot_general(oh, e1_ref[...], dn, preferred_element_type=jnp.float32)
    r2 = lax.dot_general(oh, e2_ref[...], dn, preferred_element_type=jnp.float32)
    contrib = jnp.concatenate([r0, r1, r2], axis=1)

    @pl.when(i == 0)
    def _():
        out_ref[...] = jnp.zeros_like(out_ref)

    out_ref[...] += contrib


@jax.jit
def _kg(attr_f, e0, e1, e2):
    return pl.pallas_call(
        _kg_body,
        grid=(N // BN,),
        in_specs=[
            pl.BlockSpec((1, 1, BN), lambda i: (i, 0, 0)),
            pl.BlockSpec((BN, D), lambda i: (i, 0)),
            pl.BlockSpec((BN, D), lambda i: (i, 0)),
            pl.BlockSpec((BN, D), lambda i: (i, 0)),
        ],
        out_specs=pl.BlockSpec((R, 3 * D), lambda i: (0, 0)),
        out_shape=jax.ShapeDtypeStruct((R, 3 * D), jnp.float32),
    )(attr_f, e0, e1, e2)


def kernel(edge_emb, W1, W2, Hin_idx, Hin_val, Hout_idx, Hout_val, edge_attr):
    rin, cin = _split(Hin_idx)
    rout, cout = _split(Hout_idx)

    ego0p = _pack(edge_emb)
    nin2, nout2 = _spmm2(ego0p.reshape(2 * N, HALF), cin, rin, Hin_val,
                         cout, rout, Hout_val)
    ego1p, emb1 = _layer(ego0p, nin2, nout2, W1)
    nin2b, nout2b = _spmm2(ego1p.reshape(2 * N, HALF), cin, rin, Hin_val,
                           cout, rout, Hout_val)
    _, emb2 = _layer(ego1p, nin2b, nout2b, W2)

    attr_f = edge_attr.astype(jnp.float32).reshape(N // BN, 1, BN)
    return _kg(attr_f, edge_emb, emb1, emb2)


# speedup vs baseline: 1.1537x; 1.0165x over previous
import functools

import jax
import jax.numpy as jnp
from jax import lax
from jax.experimental import pallas as pl
from jax.experimental.pallas import tpu as pltpu
from jax.experimental.pallas import tpu_sc as plsc

N = 50000
E = 800000
D = 64
R = 10
HALF = 32

NS = 16
SUB = 80
NSUB = 5
C = SUB * NSUB
EPT = E // NS
NCH = EPT // C
RPT = N // NS
ZR = 25


def _spmm_body(x2, cin, rin, vin, cout, rout, vout, nin2, nout2,
               acc, cv, gv, vv, rv, G, zbuf, sem_in, sem_g, sem_s):
    core = lax.axis_index("c")
    sid = lax.axis_index("s")

    def _zb(r, _):
        z16 = jnp.zeros((16,), jnp.float32)
        zbuf[r, pl.ds(0, 16)] = z16
        zbuf[r, pl.ds(16, 16)] = z16
        return 0
    lax.fori_loop(0, ZR, _zb, 0)

    def run_pass(cols_hbm, rows_hbm, vals_hbm, out_hbm):
        def _zz(zg, _):
            cps = [pltpu.async_copy(
                zbuf, acc.at[pl.ds(sid * RPT + (zg * 5 + u) * ZR, ZR)], sem_s)
                for u in range(5)]
            for cp in cps:
                cp.wait()
            return 0
        lax.fori_loop(0, RPT // ZR // 5, _zz, 0)
        plsc.subcore_barrier()

        def issue_in(k, p, r):
            ebase = sid * EPT + k * C
            pltpu.async_copy(cols_hbm.at[pl.ds(ebase, C)], cv.at[p], sem_in)
            pltpu.async_copy(vals_hbm.at[pl.ds(ebase, C)], vv.at[p], sem_in)
            for j in range(NSUB):
                pltpu.async_copy(rows_hbm.at[pl.ds(ebase + j * SUB, SUB)],
                                 rv.at[r, j], sem_in)

        def wait_in(p, r):
            pltpu.make_async_copy(cols_hbm.at[pl.ds(0, C)],
                                  cv.at[p], sem_in).wait()
            pltpu.make_async_copy(vals_hbm.at[pl.ds(0, C)],
                                  vv.at[p], sem_in).wait()
            for j in range(NSUB):
                pltpu.make_async_copy(rows_hbm.at[pl.ds(0, SUB)],
                                      rv.at[r, j], sem_in).wait()

        def compute_gidx(p):
            def _gi(f5, _):
                for g5 in range(5):
                    o = f5 * 80 + g5 * 16
                    c16 = cv[p, pl.ds(o, 16)]
                    gv[p, pl.ds(o, 16)] = c16 + c16 + core
                return 0
            lax.fori_loop(0, C // 80, _gi, 0)

        def issue_gather(p):
            for j in range(NSUB):
                pltpu.async_copy(x2.at[gv.at[p, pl.ds(j * SUB, SUB)]],
                                 G.at[p, pl.ds(j * SUB, SUB)], sem_g)

        def wait_gather(p):
            for j in range(NSUB):
                pltpu.make_async_copy(
                    x2.at[gv.at[p, pl.ds(j * SUB, SUB)]],
                    G.at[p, pl.ds(j * SUB, SUB)], sem_g).wait()

        def scale(p):
            def _sc(f5, _):
                for g5 in range(5):
                    o = f5 * 80 + g5 * 16
                    v16 = vv[p, pl.ds(o, 16)]
                    for e in range(16):
                        row = o + e
                        s = v16[e]
                        G[p, row, pl.ds(0, 16)] = G[p, row, pl.ds(0, 16)] * s
                        G[p, row, pl.ds(16, 16)] = G[p, row, pl.ds(16, 16)] * s
                return 0
            lax.fori_loop(0, C // 80, _sc, 0)

        def issue_scatter(p, r):
            for j in range(NSUB):
                pltpu.async_copy(G.at[p, pl.ds(j * SUB, SUB)],
                                 acc.at[rv.at[r, j]], sem_s, add=True)

        def drain_scatter(p, r):
            for j in range(NSUB):
                pltpu.make_async_copy(G.at[p, pl.ds(j * SUB, SUB)],
                                      acc.at[rv.at[r, j]], sem_s).wait()

        issue_in(0, 0, 0)
        wait_in(0, 0)
        compute_gidx(0)
        issue_gather(0)
        issue_in(1, 1, 1)

        def quad(i4, _):
            for b4 in range(4):
                k = i4 * 4 + b4
                p = b4 % 2
                pn = 1 - p
                rn = (b4 + 1) % 4
                rp = (b4 - 1) % 4
                r2 = (b4 + 2) % 4
                wait_gather(p)

                @pl.when(k < NCH - 1)
                def _():
                    wait_in(pn, rn)
                    compute_gidx(pn)

                @pl.when(k > 0)
                def _():
                    drain_scatter(pn, rp)

                @pl.when(k < NCH - 1)
                def _():
                    issue_gather(pn)

                scale(p)
                issue_scatter(p, b4)

                @pl.when(k < NCH - 2)
                def _():
                    issue_in(k + 2, p, r2)
            return 0

        lax.fori_loop(0, NCH // 4, quad, 0)
        wait_gather(0)
        drain_scatter(1, 3)
        scale(0)
        issue_scatter(0, 0)
        drain_scatter(0, 0)

        plsc.subcore_barrier()
        pltpu.sync_copy(acc.at[pl.ds(sid * RPT, RPT)],
                        out_hbm.at[core, pl.ds(sid * RPT, RPT)])
        plsc.subcore_barrier()

    run_pass(cin, rin, vin, nin2)
    run_pass(cout, rout, vout, nout2)


@jax.jit
def _spmm2(x2, cin, rin, vin, cout, rout, vout):
    mesh = plsc.VectorSubcoreMesh(core_axis_name="c", subcore_axis_name="s")
    f = pl.kernel(
        _spmm_body,
        out_type=(jax.ShapeDtypeStruct((2, N, HALF), jnp.float32),
                  jax.ShapeDtypeStruct((2, N, HALF), jnp.float32)),
        mesh=mesh,
        scratch_types=(
            pltpu.VMEM_SHARED((N, HALF), jnp.float32),
            pltpu.VMEM((2, C), jnp.int32),
            pltpu.VMEM((2, C), jnp.int32),
            pltpu.VMEM((2, C), jnp.float32),
            pltpu.VMEM((4, NSUB, SUB), jnp.int32),
            pltpu.VMEM((2, C, HALF), jnp.float32),
            pltpu.VMEM((ZR, HALF), jnp.float32),
            pltpu.SemaphoreType.DMA,
            pltpu.SemaphoreType.DMA,
            pltpu.SemaphoreType.DMA,
        ),
        compiler_params=pltpu.CompilerParams(use_tc_tiling_on_sc=False),
    )
    return f(x2, cin, rin, vin, cout, rout, vout)


BN = 2000


BX = 80000


def _split_body(idx_ref, r_ref, c_ref):
    r_ref[...] = idx_ref[0]
    c_ref[...] = idx_ref[1]


@jax.jit
def _split(idx):
    return pl.pallas_call(
        _split_body,
        grid=(1,),
        in_specs=[pl.BlockSpec((2, E), lambda i: (0, 0))],
        out_specs=[
            pl.BlockSpec((E,), lambda i: (0,)),
            pl.BlockSpec((E,), lambda i: (0,)),
        ],
        out_shape=[
            jax.ShapeDtypeStruct((E,), jnp.int32),
            jax.ShapeDtypeStruct((E,), jnp.int32),
        ],
    )(idx)


def _onehot(attr_ref):
    a = attr_ref[...].reshape(BN, 1)
    lbl = lax.broadcasted_iota(jnp.int32, (BN, R), 1).astype(jnp.float32)
    return (a == lbl).astype(jnp.float32)


_DN = (((0,), (0,)), ((), ()))


def _pack_body(x_ref, attr_ref, o_ref, kg_ref):
    i = pl.program_id(0)
    x = x_ref[...]
    o_ref[:, 0:D] = x_ref[pl.ds(0, BN // 2, 2), :]
    o_ref[:, D:2 * D] = x_ref[pl.ds(1, BN // 2, 2), :]
    oh = _onehot(attr_ref)

    @pl.when(i == 0)
    def _():
        kg_ref[...] = jnp.zeros_like(kg_ref)

    kg_ref[...] += lax.dot_general(oh, x, _DN,
                                   preferred_element_type=jnp.float32)


@jax.jit
def _pack(x, attr_f):
    return pl.pallas_call(
        _pack_body,
        grid=(N // BN,),
        in_specs=[pl.BlockSpec((BN, D), lambda i: (i, 0)),
                  pl.BlockSpec((1, 1, BN), lambda i: (i, 0, 0))],
        out_specs=[pl.BlockSpec((BN // 2, 2 * D), lambda i: (i, 0)),
                   pl.BlockSpec((R, D), lambda i: (0, 0))],
        out_shape=[jax.ShapeDtypeStruct((N // 2, 2 * D), jnp.float32),
                   jax.ShapeDtypeStruct((R, D), jnp.float32)],
    )(x, attr_f)


def _layer_body(ego_ref, nin_ref, nout_ref, w_ref, attr_ref, ego_o_ref,
                kg_ref, scr_ref):
    i = pl.program_id(0)
    scr_ref[pl.ds(0, BN // 2, 2), :] = ego_ref[:, 0:D]
    scr_ref[pl.ds(1, BN // 2, 2), :] = ego_ref[:, D:2 * D]
    ego = scr_ref[...]
    w = w_ref[...]
    acc = jnp.dot(ego, w[0:64], preferred_element_type=jnp.float32)
    acc = acc + jnp.dot(nin_ref[0], w[64:96], preferred_element_type=jnp.float32)
    acc = acc + jnp.dot(nin_ref[1], w[96:128], preferred_element_type=jnp.float32)
    acc = acc + jnp.dot(nout_ref[0], w[128:160], preferred_element_type=jnp.float32)
    acc = acc + jnp.dot(nout_ref[1], w[160:192], preferred_element_type=jnp.float32)
    ego_n = jnp.where(acc >= 0, acc, 0.01 * acc)
    scr_ref[...] = ego_n
    ego_o_ref[:, 0:D] = scr_ref[pl.ds(0, BN // 2, 2), :]
    ego_o_ref[:, D:2 * D] = scr_ref[pl.ds(1, BN // 2, 2), :]
    nrm = jnp.sqrt(jnp.sum(ego_n * ego_n, axis=1, keepdims=True))
    emb = ego_n / jnp.maximum(nrm, 1e-12)
    oh = _onehot(attr_ref)

    @pl.when(i == 0)
    def _():
        kg_ref[...] = jnp.zeros_like(kg_ref)

    kg_ref[...] += lax.dot_general(oh, emb, _DN,
                                   preferred_element_type=jnp.float32)


@jax.jit
def _layer(ego_p, nin2, nout2, w, attr_f):
    return pl.pallas_call(
        _layer_body,
        grid=(N // BN,),
        in_specs=[
            pl.BlockSpec((BN // 2, 2 * D), lambda i: (i, 0)),
            pl.BlockSpec((2, BN, HALF), lambda i: (0, i, 0)),
            pl.BlockSpec((2, BN, HALF), lambda i: (0, i, 0)),
            pl.BlockSpec((3 * D, D), lambda i: (0, 0)),
            pl.BlockSpec((1, 1, BN), lambda i: (i, 0, 0)),
        ],
        out_specs=[
            pl.BlockSpec((BN // 2, 2 * D), lambda i: (i, 0)),
            pl.BlockSpec((R, D), lambda i: (0, 0)),
        ],
        out_shape=[
            jax.ShapeDtypeStruct((N // 2, 2 * D), jnp.float32),
            jax.ShapeDtypeStruct((R, D), jnp.float32),
        ],
        scratch_shapes=[pltpu.VMEM((BN, D), jnp.float32)],
    )(ego_p, nin2, nout2, w, attr_f)


def kernel(edge_emb, W1, W2, Hin_idx, Hin_val, Hout_idx, Hout_val, edge_attr):
    rin, cin = _split(Hin_idx)
    rout, cout = _split(Hout_idx)
    attr_f = edge_attr.astype(jnp.float32).reshape(N // BN, 1, BN)

    ego0p, kg0 = _pack(edge_emb, attr_f)
    nin2, nout2 = _spmm2(ego0p.reshape(2 * N, HALF), cin, rin, Hin_val,
                         cout, rout, Hout_val)
    ego1p, kg1 = _layer(ego0p, nin2, nout2, W1, attr_f)
    nin2b, nout2b = _spmm2(ego1p.reshape(2 * N, HALF), cin, rin, Hin_val,
                           cout, rout, Hout_val)
    _, kg2 = _layer(ego1p, nin2b, nout2b, W2, attr_f)

    return jnp.concatenate([kg0, kg1, kg2], axis=1)
